# Initial kernel scaffold; baseline (speedup 1.0000x reference)
import jax
import jax.numpy as jnp
from jax.experimental import pallas as pl


def kernel(x):
    raise NotImplementedError("write your pallas kernel here")


# speedup vs baseline: 1.5204x; 1.5204x over previous
import functools

import jax
import jax.numpy as jnp
from jax.experimental import pallas as pl
from jax.experimental.pallas import tpu as pltpu


def _wb_body(x_ref, o_ref, *, inv_n):
    r = x_ref[0, 0]
    g = x_ref[0, 1]
    b = x_ref[0, 2]
    g_avg = jnp.sum(g) * inv_n
    r_gain = g_avg / (jnp.sum(r) * inv_n + 1e-6)
    g_gain = g_avg / (g_avg + 1e-6)
    b_gain = g_avg / (jnp.sum(b) * inv_n + 1e-6)
    o_ref[0, 0] = r * r_gain
    o_ref[0, 1] = g * g_gain
    o_ref[0, 2] = b * b_gain


def kernel(x):
    B, C, H, W = x.shape
    assert C == 3, "gray-world RGB path expects 3 channels"
    blk = (1, C, H, W)
    body = functools.partial(_wb_body, inv_n=1.0 / (H * W))
    n_bytes = 2 * int(x.size) * x.dtype.itemsize
    return pl.pallas_call(
        body,
        out_shape=jax.ShapeDtypeStruct(x.shape, x.dtype),
        grid=(B,),
        in_specs=[pl.BlockSpec(blk, lambda i: (i, 0, 0, 0))],
        out_specs=pl.BlockSpec(blk, lambda i: (i, 0, 0, 0)),
        compiler_params=pltpu.CompilerParams(
            dimension_semantics=("parallel",),
            vmem_limit_bytes=60 * 1024 * 1024,
        ),
        cost_estimate=pl.CostEstimate(
            flops=3 * int(x.size), transcendentals=0, bytes_accessed=n_bytes),
    )(x)
